# Initial kernel scaffold; baseline (speedup 1.0000x reference)
import jax
import jax.numpy as jnp
from jax.experimental import pallas as pl


def kernel(x_team, x_player, ei_win, ei_loss, ei_tie, ei_team_before, ei_team_after, ei_playedin, ei_used, ei_player_before, ei_player_after, home_list, away_list, emb_table, W_gcn, b_gcn, W_gat_src, W_gat_dst, a_gat_src, a_gat_dst, b_gat, W_fc1, b_fc1, W_fc2, b_fc2):
    raise NotImplementedError("write your pallas kernel here")


# speedup vs baseline: 1.1186x; 1.1186x over previous
import functools

import jax
import jax.numpy as jnp
from jax.experimental import pallas as pl


def _mm_kernel(x_ref, w_ref, o_ref):
    o_ref[...] = jnp.dot(x_ref[...], w_ref[...],
                         preferred_element_type=jnp.float32)


@functools.partial(jax.jit, static_argnames=("block_rows",))
def _mm(x, w, block_rows):
    n, k = x.shape
    m = w.shape[1]
    grid = (n // block_rows,)
    return pl.pallas_call(
        _mm_kernel,
        grid=grid,
        in_specs=[
            pl.BlockSpec((block_rows, k), lambda i: (i, 0)),
            pl.BlockSpec((k, m), lambda i: (0, 0)),
        ],
        out_specs=pl.BlockSpec((block_rows, m), lambda i: (i, 0)),
        out_shape=jax.ShapeDtypeStruct((n, m), jnp.float32),
    )(x, w)


def _head_kernel(h_ref, w1_ref, b1_ref, w2_ref, b2_ref, o_ref):
    h1 = jnp.maximum(
        jnp.dot(h_ref[...], w1_ref[...], preferred_element_type=jnp.float32)
        + b1_ref[...], 0.0)
    logits = jnp.dot(h1, w2_ref[...],
                     preferred_element_type=jnp.float32) + b2_ref[...]
    valid = jax.lax.broadcasted_iota(jnp.int32, logits.shape, 1) < 2
    neg = jnp.full_like(logits, -jnp.inf)
    masked = jnp.where(valid, logits, neg)
    mx = jnp.max(masked, axis=1, keepdims=True)
    lse = mx + jnp.log(
        jnp.sum(jnp.where(valid, jnp.exp(masked - mx), 0.0), axis=1,
                keepdims=True))
    o_ref[...] = logits - lse


@jax.jit
def _head(h, w1, b1, w2p, b2p):
    n = h.shape[0]
    br = 512
    return pl.pallas_call(
        _head_kernel,
        grid=(n // br,),
        in_specs=[
            pl.BlockSpec((br, h.shape[1]), lambda i: (i, 0)),
            pl.BlockSpec(w1.shape, lambda i: (0, 0)),
            pl.BlockSpec((1, w1.shape[1]), lambda i: (0, 0)),
            pl.BlockSpec(w2p.shape, lambda i: (0, 0)),
            pl.BlockSpec((1, w2p.shape[1]), lambda i: (0, 0)),
        ],
        out_specs=pl.BlockSpec((br, w2p.shape[1]), lambda i: (i, 0)),
        out_shape=jax.ShapeDtypeStruct((n, w2p.shape[1]), jnp.float32),
    )(h, w1, b1.reshape(1, -1), w2p, b2p.reshape(1, -1))


def _gcn(xw, ei, b, n):
    row, col = ei[0], ei[1]
    deg = jnp.zeros((n,), jnp.float32).at[col].add(1.0) + 1.0
    dis = jax.lax.rsqrt(deg)
    norm = dis[row] * dis[col]
    out = jnp.zeros((n, xw.shape[1]), jnp.float32).at[col].add(
        norm[:, None] * xw[row])
    out = out + (dis * dis)[:, None] * xw
    return out + b


def _gat(hs, hd, s_src, s_dst, ei, b, n_dst):
    row, col = ei[0], ei[1]
    e = jax.nn.leaky_relu(s_src[row] + s_dst[col], 0.2)
    m = jax.ops.segment_max(e, col, num_segments=n_dst)
    m = jnp.where(jnp.isfinite(m), m, 0.0)
    ex = jnp.exp(e - m[col])
    den = jax.ops.segment_sum(ex, col, num_segments=n_dst)
    coef = ex / (den[col] + 1e-16)
    out = jnp.zeros((n_dst, hs.shape[1]), jnp.float32).at[col].add(
        coef[:, None] * hs[row])
    return out + b


def _layer(xt, xp, eis, Wg, bg, Wgs, Wgd, ags, agd, bgt):
    ei_win, ei_loss, ei_tie, ei_tb, ei_ta, ei_pi, ei_us, ei_pb, ei_pa = eis
    nt, npl = xt.shape[0], xp.shape[0]
    D = xt.shape[1]

    sc_pi_dst = (Wgd[0] @ agd[0])[:, None]
    sc_us_src = (Wgs[1] @ ags[1])[:, None]
    wt = jnp.concatenate(
        [Wg[0], Wg[1], Wg[2], Wg[3], Wg[4], Wgd[0], Wgs[1],
         sc_pi_dst, sc_us_src, jnp.zeros((D, 126), jnp.float32)], axis=1)
    yt = _mm(xt, wt, 1000)

    sc_pi_src = (Wgs[0] @ ags[0])[:, None]
    sc_us_dst = (Wgd[1] @ agd[1])[:, None]
    wp = jnp.concatenate(
        [Wg[5], Wg[6], Wgs[0], Wgd[1],
         sc_pi_src, sc_us_dst, jnp.zeros((D, 126), jnp.float32)], axis=1)
    yp = _mm(xp, wp, 1000)

    xw_win, xw_loss, xw_tie, xw_tb, xw_ta = (
        yt[:, 0:D], yt[:, D:2 * D], yt[:, 2 * D:3 * D], yt[:, 3 * D:4 * D],
        yt[:, 4 * D:5 * D])
    hd_pi = yt[:, 5 * D:6 * D]
    hs_us = yt[:, 6 * D:7 * D]
    sdst_pi = yt[:, 7 * D]
    ssrc_us = yt[:, 7 * D + 1]

    xw_pb, xw_pa = yp[:, 0:D], yp[:, D:2 * D]
    hs_pi = yp[:, 2 * D:3 * D]
    hd_us = yp[:, 3 * D:4 * D]
    ssrc_pi = yp[:, 4 * D]
    sdst_us = yp[:, 4 * D + 1]

    out_t = (_gcn(xw_win, ei_win, bg[0], nt)
             + _gcn(xw_loss, ei_loss, bg[1], nt)
             + _gcn(xw_tie, ei_tie, bg[2], nt)
             + _gcn(xw_tb, ei_tb, bg[3], nt)
             + _gcn(xw_ta, ei_ta, bg[4], nt)
             + _gat(hs_pi, hd_pi, ssrc_pi, sdst_pi, ei_pi, bgt[0], nt))
    out_p = (_gat(hs_us, hd_us, ssrc_us, sdst_us, ei_us, bgt[1], npl)
             + _gcn(xw_pb, ei_pb, bg[5], npl)
             + _gcn(xw_pa, ei_pa, bg[6], npl))
    return out_t, out_p


def kernel(x_team, x_player, ei_win, ei_loss, ei_tie, ei_team_before,
           ei_team_after, ei_playedin, ei_used, ei_player_before,
           ei_player_after, home_list, away_list, emb_table, W_gcn, b_gcn,
           W_gat_src, W_gat_dst, a_gat_src, a_gat_dst, b_gat, W_fc1, b_fc1,
           W_fc2, b_fc2):
    eis = (ei_win, ei_loss, ei_tie, ei_team_before, ei_team_after,
           ei_playedin, ei_used, ei_player_before, ei_player_after)
    xt = jnp.take(emb_table, x_team, axis=0)
    xp = jnp.take(emb_table, x_player, axis=0)
    t, p = _layer(xt, xp, eis, W_gcn[0], b_gcn[0], W_gat_src[0], W_gat_dst[0],
                  a_gat_src[0], a_gat_dst[0], b_gat[0])
    t = jnp.maximum(t, 0.0)
    p = jnp.maximum(p, 0.0)
    t, p = _layer(t, p, eis, W_gcn[1], b_gcn[1], W_gat_src[1], W_gat_dst[1],
                  a_gat_src[1], a_gat_dst[1], b_gat[1])
    h = jnp.concatenate(
        [jnp.take(t, home_list, axis=0), jnp.take(t, away_list, axis=0)],
        axis=1)
    w2p = jnp.concatenate(
        [W_fc2, jnp.zeros((W_fc2.shape[0], 126), jnp.float32)], axis=1)
    b2p = jnp.concatenate([b_fc2, jnp.zeros((126,), jnp.float32)])
    out = _head(h, W_fc1, b_fc1, w2p, b2p)
    return out[:, :2]
